# baseline (speedup 1.0000x reference)
import functools

import jax
import jax.numpy as jnp
from jax import lax
from jax.experimental import pallas as pl
from jax.experimental.pallas import tpu as pltpu
from jax.experimental.pallas import tpu_sc as plsc

B = 16
IMG = 224
PATCH = 16
FEAT = 768
HID = 768
EMB = 384
K = 2048
BETA = 0.25
EPS = 1e-8

N = 2 * B * 14 * 14
HALF = N // 2
T = 784
TL = 896
P = 392
NT = N // T
NL = N // TL
NP = HALF // P
NPAD = 6400


def _leaky(x):
    return jnp.where(x >= 0, x, 0.1 * x)


def _rnd(x):
    return x.astype(jnp.bfloat16).astype(jnp.float32)


def _dot(a, b):
    return jnp.dot(a, b, preferred_element_type=jnp.float32)


def _resblock_cm(x, w1, b1, w2, b2):
    h = _rnd(_leaky(_dot(w1, x) + b1))
    return x + (_dot(w2, h) + b2)


def _k1_kern(x_ref, pw, pb, e0w1, e0b1, e0w2, e0b2, e1w1, e1b1, e1w2, e1b2,
             vqw, dino_ref, feat_ref, f0_ref):
    dino = _dot(pw[...], x_ref[...]) + pb[...]
    t = _resblock_cm(dino, e0w1[...], e0b1[...], e0w2[...], e0b2[...])
    feat = _resblock_cm(t, e1w1[...], e1b1[...], e1w2[...], e1b2[...])
    dino_ref[...] = dino
    feat_ref[...] = feat
    f0_ref[...] = _dot(vqw[...], _rnd(_leaky(feat)))


def _vq_kern(za_ref, zb_ref, cbT_ref, c2_ref, ia_ref, ib_ref, jsd_ref, loss_ref):
    i = pl.program_id(0)
    cbT = cbT_ref[...]
    c2 = c2_ref[...]

    def half(z):
        z2 = jnp.sum(z * z, axis=1, keepdims=True)
        d = z2 + c2 - 2.0 * _dot(z, cbT)
        dmin = jnp.min(d, axis=1, keepdims=True)
        lanes = jax.lax.broadcasted_iota(jnp.int32, d.shape, 1)
        idx = jnp.min(jnp.where(d == dmin, lanes, K), axis=1)
        e = jnp.exp(dmin - d)
        p = e / jnp.sum(e, axis=1, keepdims=True)
        return p, idx, dmin

    pa, idxa, dmina = half(za_ref[...])
    pb, idxb, dminb = half(zb_ref[...])
    ia_ref[...] = idxa[:, None]
    ib_ref[...] = idxb[:, None]

    m = 0.5 * (pa + pb)
    lm = jnp.log(m + EPS)
    kl1 = jnp.sum(pa * (jnp.log(pa + EPS) - lm), axis=1)
    kl2 = jnp.sum(pb * (jnp.log(pb + EPS) - lm), axis=1)
    jsd_part = 0.5 * jnp.sum(kl1 + kl2)
    loss_part = jnp.sum(dmina) + jnp.sum(dminb)

    @pl.when(i == 0)
    def _():
        jsd_ref[...] = jnp.zeros_like(jsd_ref)
        loss_ref[...] = jnp.zeros_like(loss_ref)

    jsd_ref[...] += jnp.reshape(jsd_part, (1, 1))
    loss_ref[...] += jnp.reshape(loss_part, (1, 1))


def _sc_gather(cb, idx, npad):
    info = plsc.get_sparse_core_info()
    nw = info.num_cores * info.num_subcores
    b_per_w = npad // nw
    mesh = plsc.VectorSubcoreMesh(core_axis_name="c", subcore_axis_name="s")

    @functools.partial(
        pl.kernel, mesh=mesh,
        out_type=jax.ShapeDtypeStruct((npad, EMB), jnp.float32),
        scratch_types=[
            pltpu.VMEM((b_per_w,), jnp.int32),
            pltpu.VMEM((b_per_w, EMB), jnp.float32),
            pltpu.SemaphoreType.DMA,
        ],
    )
    def k(table_hbm, idx_hbm, out_hbm, idx_v, rows_v, sem):
        wid = lax.axis_index("s") * info.num_cores + lax.axis_index("c")
        base = wid * b_per_w
        pltpu.sync_copy(idx_hbm.at[pl.ds(base, b_per_w)], idx_v)
        pltpu.async_copy(table_hbm.at[idx_v], rows_v, sem).wait()
        pltpu.sync_copy(rows_v, out_hbm.at[pl.ds(base, b_per_w)])

    return k(cb, idx)


def _k3_kern(feat_ref, q0_ref, w, bias, vqw, f1_ref):
    cat = jnp.concatenate([feat_ref[...], q0_ref[...]], axis=0)
    feat2 = _dot(w[...], cat) + bias[...]
    f1_ref[...] = _dot(vqw[...], _rnd(_leaky(feat2)))


def _k5_kern(q0_ref, q1_ref, dino_ref, w, bias,
             d0w1, d0b1, d0w2, d0b2, d1w1, d1b1, d1w2, d1b2,
             feat_ref, rec_ref):
    i = pl.program_id(0)
    cat = jnp.concatenate([q0_ref[...], q1_ref[...]], axis=0)
    f = _dot(w[...], cat) + bias[...]
    r = _resblock_cm(f, d0w1[...], d0b1[...], d0w2[...], d0b2[...])
    r = _resblock_cm(r, d1w1[...], d1b1[...], d1w2[...], d1b2[...])
    feat_ref[...] = f
    part = jnp.sum((r - dino_ref[...]) ** 2)

    @pl.when(i == 0)
    def _():
        rec_ref[...] = jnp.zeros_like(rec_ref)

    rec_ref[...] += jnp.reshape(part, (1, 1))


def _full(shape):
    return pl.BlockSpec(shape, lambda i: (0, 0))


def _rows(bs, c):
    return pl.BlockSpec((bs, c), lambda i: (i, 0))


def _cols(c, bs):
    return pl.BlockSpec((c, bs), lambda i: (0, i))


def _scalar():
    return pl.BlockSpec((1, 1), lambda i: (0, 0))


def _run_vq(f, cb):
    cbT = cb.T
    c2 = jnp.sum(cb * cb, axis=1).reshape(1, K)
    idxa, idxb, jsd_sum, loss_sum = pl.pallas_call(
        _vq_kern,
        grid=(NP,),
        in_specs=[
            pl.BlockSpec((P, EMB), lambda i: (i, 0)),
            pl.BlockSpec((P, EMB), lambda i: (i + NP, 0)),
            _full((EMB, K)),
            _full((1, K)),
        ],
        out_specs=[_rows(P, 1), _rows(P, 1), _scalar(), _scalar()],
        out_shape=[
            jax.ShapeDtypeStruct((HALF, 1), jnp.int32),
            jax.ShapeDtypeStruct((HALF, 1), jnp.int32),
            jax.ShapeDtypeStruct((1, 1), jnp.float32),
            jax.ShapeDtypeStruct((1, 1), jnp.float32),
        ],
    )(f, f, cbT, c2)
    idx = jnp.concatenate([idxa.reshape(HALF), idxb.reshape(HALF),
                           jnp.zeros((NPAD - N,), jnp.int32)])
    q = _sc_gather(cb, idx, NPAD)[:N]
    jsd = jsd_sum[0, 0] / HALF
    loss = (1.0 + BETA) * loss_sum[0, 0] / (N * EMB)
    return q, jsd, loss


def kernel(img, params):
    p = params
    ka, kb = jax.random.split(jax.random.key(1234))
    scale = jax.random.uniform(ka, (B, 3, 1, 1), jnp.float32, 0.9, 1.1)
    off = jax.random.uniform(kb, (B, 3, 1, 1), jnp.float32, -0.1, 0.1)
    x = jnp.concatenate([img, img * scale + off], axis=0)

    patchesT = (x.reshape(2 * B, 3, 14, PATCH, 14, PATCH)
                 .transpose(1, 3, 5, 0, 2, 4)
                 .reshape(3 * PATCH * PATCH, N))

    col = lambda b: b.reshape(-1, 1)

    dinoT, featT, f0T = pl.pallas_call(
        _k1_kern,
        grid=(NL,),
        in_specs=[
            _cols(768, TL),
            _full((FEAT, 768)), _full((FEAT, 1)),
            _full((HID, HID)), _full((HID, 1)), _full((HID, HID)), _full((HID, 1)),
            _full((HID, HID)), _full((HID, 1)), _full((HID, HID)), _full((HID, 1)),
            _full((EMB, HID)),
        ],
        out_specs=[_cols(FEAT, TL), _cols(HID, TL), _cols(EMB, TL)],
        out_shape=[
            jax.ShapeDtypeStruct((FEAT, N), jnp.float32),
            jax.ShapeDtypeStruct((HID, N), jnp.float32),
            jax.ShapeDtypeStruct((EMB, N), jnp.float32),
        ],
    )(patchesT,
      p['pe_w'].reshape(FEAT, 768), col(p['pe_b']),
      p['enc0_w1'], col(p['enc0_b1']), p['enc0_w2'], col(p['enc0_b2']),
      p['enc1_w1'], col(p['enc1_b1']), p['enc1_w2'], col(p['enc1_b2']),
      p['vq0_in_w'])

    q0, jsd0, l0 = _run_vq(f0T.T, p['cb0'])
    q0T = q0.T

    f1T = pl.pallas_call(
        _k3_kern,
        grid=(NL,),
        in_specs=[
            _cols(HID, TL), _cols(EMB, TL),
            _full((HID, HID + EMB)), _full((HID, 1)),
            _full((EMB, HID)),
        ],
        out_specs=_cols(EMB, TL),
        out_shape=jax.ShapeDtypeStruct((EMB, N), jnp.float32),
    )(featT, q0T, p['vqout0_w'], col(p['vqout0_b']), p['vq1_in_w'])

    q1, jsd1, l1 = _run_vq(f1T.T, p['cb1'])
    q1T = q1.T

    feat3T, rec_sum = pl.pallas_call(
        _k5_kern,
        grid=(NL,),
        in_specs=[
            _cols(EMB, TL), _cols(EMB, TL), _cols(FEAT, TL),
            _full((HID, 2 * EMB)), _full((HID, 1)),
            _full((HID, HID)), _full((HID, 1)), _full((HID, HID)), _full((HID, 1)),
            _full((HID, HID)), _full((HID, 1)), _full((HID, HID)), _full((HID, 1)),
        ],
        out_specs=[_cols(HID, TL), _scalar()],
        out_shape=[
            jax.ShapeDtypeStruct((HID, N), jnp.float32),
            jax.ShapeDtypeStruct((1, 1), jnp.float32),
        ],
    )(q0T, q1T, dinoT, p['agg_w'], col(p['agg_b']),
      p['dec0_w1'], col(p['dec0_b1']), p['dec0_w2'], col(p['dec0_b2']),
      p['dec1_w1'], col(p['dec1_b1']), p['dec1_w2'], col(p['dec1_b2']))

    feat_out = feat3T.reshape(HID, 2 * B, 14, 14)[:, :B].transpose(1, 0, 2, 3)
    untok = lambda a, c: a[:HALF].reshape(B, 14, 14, c).transpose(0, 3, 1, 2)
    recon_loss = rec_sum[0, 0] / (N * FEAT)
    contra_loss = jsd0 - 0.1 * jsd1
    return (feat_out, untok(q0, EMB), untok(q1, EMB),
            recon_loss, contra_loss, l0 + l1)
